# Initial kernel scaffold; baseline (speedup 1.0000x reference)
import jax
import jax.numpy as jnp
from jax.experimental import pallas as pl


def kernel(x, W, b):
    raise NotImplementedError("write your pallas kernel here")


# speedup vs baseline: 1.2282x; 1.2282x over previous
import functools

import jax
import jax.numpy as jnp
from jax.experimental import pallas as pl

T = 32768
D = 1024
E = 64
K = 8
BLK = 2048


def _router_block(x_ref, wt_ref, b_ref, idx_ref, w_ref):
    x = x_ref[...]
    wt = wt_ref[...]
    logits = jax.lax.dot_general(
        x, wt, (((1,), (0,)), ((), ())), preferred_element_type=jnp.float32
    ) + b_ref[...]
    m = jnp.max(logits, axis=1, keepdims=True)
    p = jnp.exp(logits - m)
    denom = jnp.sum(p, axis=1, keepdims=True)
    lane = jax.lax.broadcasted_iota(jnp.int32, (BLK, E), 1)
    idx_cols = []
    val_cols = []
    for _ in range(K):
        v = jnp.max(p, axis=1, keepdims=True)
        i = jnp.min(jnp.where(p == v, lane, E), axis=1, keepdims=True)
        idx_cols.append(i)
        val_cols.append(v)
        p = jnp.where(lane == i, -1.0, p)
    idx_ref[...] = jnp.concatenate(idx_cols, axis=1)
    w_ref[...] = jnp.concatenate(val_cols, axis=1) / denom


@jax.jit
def kernel(x, W, b):
    wt = W.T
    b2 = b.reshape(1, E)
    grid = (T // BLK,)
    return pl.pallas_call(
        _router_block,
        grid=grid,
        in_specs=[
            pl.BlockSpec((BLK, D), lambda i: (i, 0)),
            pl.BlockSpec((D, E), lambda i: (0, 0)),
            pl.BlockSpec((1, E), lambda i: (0, 0)),
        ],
        out_specs=[
            pl.BlockSpec((BLK, K), lambda i: (i, 0)),
            pl.BlockSpec((BLK, K), lambda i: (i, 0)),
        ],
        out_shape=[
            jax.ShapeDtypeStruct((T, K), jnp.int32),
            jax.ShapeDtypeStruct((T, K), jnp.float32),
        ],
    )(x, wt, b2)


# speedup vs baseline: 1.5356x; 1.2503x over previous
import jax
import jax.numpy as jnp
from jax.experimental import pallas as pl

T = 32768
D = 1024
E = 64
K = 8
BLK = 2048
INT_MIN = -2147483648
MASK7F = 0x7FFFFFFF


def _router_block(x_ref, wt_ref, b_ref, idx_ref, w_ref):
    x = x_ref[...]
    wt = wt_ref[...]
    logits = jax.lax.dot_general(
        x, wt, (((1,), (0,)), ((), ())), preferred_element_type=jnp.float32
    ) + b_ref[...]
    m0 = jnp.max(logits, axis=1, keepdims=True)
    denom = jnp.sum(jnp.exp(logits - m0), axis=1, keepdims=True)
    lane = jax.lax.broadcasted_iota(jnp.int32, (BLK, E), 1)
    li = jax.lax.bitcast_convert_type(logits, jnp.int32)
    key = jnp.where(li >= 0, li, li ^ jnp.int32(MASK7F))
    key = (key & jnp.int32(-64)) | (jnp.int32(E - 1) - lane)
    best = []
    for _ in range(K):
        mk = jnp.max(key, axis=1, keepdims=True)
        best.append(mk)
        key = jnp.where(key == mk, jnp.int32(INT_MIN), key)
    mks = jnp.concatenate(best, axis=1)
    idx_ref[...] = jnp.int32(E - 1) - (mks & jnp.int32(E - 1))
    dec = jnp.where(mks >= 0, mks, mks ^ jnp.int32(MASK7F))
    logit_k = jax.lax.bitcast_convert_type(dec, jnp.float32)
    w_ref[...] = jnp.exp(logit_k - m0) / denom


@jax.jit
def kernel(x, W, b):
    wt = W.T
    b2 = b.reshape(1, E)
    grid = (T // BLK,)
    return pl.pallas_call(
        _router_block,
        grid=grid,
        in_specs=[
            pl.BlockSpec((BLK, D), lambda i: (i, 0)),
            pl.BlockSpec((D, E), lambda i: (0, 0)),
            pl.BlockSpec((1, E), lambda i: (0, 0)),
        ],
        out_specs=[
            pl.BlockSpec((BLK, K), lambda i: (i, 0)),
            pl.BlockSpec((BLK, K), lambda i: (i, 0)),
        ],
        out_shape=[
            jax.ShapeDtypeStruct((T, K), jnp.int32),
            jax.ShapeDtypeStruct((T, K), jnp.float32),
        ],
    )(x, wt, b2)
